# baseline (speedup 1.0000x reference)
import jax
import jax.numpy as jnp
from jax import lax
from jax.experimental import pallas as pl
from jax.experimental.pallas import tpu as pltpu
from jax.experimental.pallas import tpu_sc as plsc

B = 16
T = 300
C = 2048
S_PAD = 2048
S_NPAD = 208
E_DEN = 16000
E_NUM = 1600
CAP_DEN = 16256
HALF = 1112
REST = 888
T_PAD = 304
NEG = -1e5
L = 16


def _sc_forward_body(ll_h, seql_h, prm_h, dsd_h, dpdf_h, dw_h,
                     nsd_h, npdf_h, nw_h, dstart_h, nstart_h,
                     ealpha_o, scales_o,
                     dsd_v, dpdf_v, dw_v, nsd_v, npdf_v, nw_v,
                     bufDA, bufDB, bufNA, bufNB, llA, llB,
                     seql_v, prm_v, scalD_v, scalN_v, invD_v, invN_v,
                     spmA, spmB, semA, semB):
    c = lax.axis_index("c")
    s = lax.axis_index("s")
    p = lax.shift_right_logical(s, 1)
    h = jnp.bitwise_and(s, 1)
    u = c * 8 + p
    is_a = h == 0
    is_b = h == 1
    lanes = lax.iota(jnp.int32, L)

    pltpu.sync_copy(seql_h, seql_v)
    pltpu.sync_copy(prm_h, prm_v)
    pltpu.sync_copy(dsd_h, dsd_v)
    pltpu.sync_copy(dpdf_h, dpdf_v)
    pltpu.sync_copy(dw_h, dw_v)
    pltpu.sync_copy(dstart_h, bufDA)

    @pl.when(is_b)
    def _():
        pltpu.sync_copy(nsd_h, nsd_v)
        pltpu.sync_copy(npdf_h, npdf_v)
        pltpu.sync_copy(nw_h, nw_v)
        pltpu.sync_copy(nstart_h, bufNA)

    seqlen = seql_v[...].at[jnp.full((L,), u, jnp.int32)].get(
        mode="promise_in_bounds")[0]
    prm = prm_v[...]
    cA = prm[0]
    startB = prm[1]
    cB = prm[2]

    @plsc.parallel_loop(0, CAP_DEN // L, unroll=4)
    def _expwd(i):
        sl = pl.ds(i * L, L)
        dw_v[sl] = jnp.exp(dw_v[sl])

    @plsc.parallel_loop(0, S_PAD // L, unroll=4)
    def _initD(i):
        sl = pl.ds(i * L, L)
        bufDA[sl] = jnp.exp(bufDA[sl])
        bufDB[sl] = jnp.zeros((L,), jnp.float32)

    @plsc.parallel_loop(0, T_PAD // L, unroll=4)
    def _ones(i):
        sl = pl.ds(i * L, L)
        scalD_v[sl] = jnp.full((L,), 1.0, jnp.float32)
        scalN_v[sl] = jnp.full((L,), 1.0, jnp.float32)

    @pl.when(is_b)
    def _():
        @plsc.parallel_loop(0, E_NUM // L, unroll=4)
        def _expwn(i):
            sl = pl.ds(i * L, L)
            nw_v[sl] = jnp.exp(nw_v[sl])

        @plsc.parallel_loop(0, S_NPAD // L)
        def _initN(i):
            sl = pl.ds(i * L, L)
            bufNA[sl] = jnp.exp(bufNA[sl])
            bufNB[sl] = jnp.zeros((L,), jnp.float32)

    invD_v[...] = jnp.full((L,), 1.0, jnp.float32)
    invN_v[...] = jnp.full((L,), 1.0, jnp.float32)
    pltpu.async_copy(ll_h.at[u, 0], llA, semA)

    def arc_pass(lo2, hi2, unroll, sdv, pdfv, wv, rbuf, wbuf, llbuf, ivec):
        @plsc.parallel_loop(lo2, hi2, unroll=unroll)
        def _arcs(i):
            base = i * 2 * L
            pp = pdfv[pl.ds(base, 2 * L)]
            p0, p1 = plsc.unpack(pp, format=plsc.PackFormat.INTERLEAVED,
                                 preferred_element_type=jnp.int32)
            for j, pj in ((0, p0), (1, p1)):
                sl = pl.ds(base + j * L, L)
                sd = sdv[sl]
                src = jnp.bitwise_and(sd, 0xFFFF)
                dst = lax.shift_right_logical(sd, 16)
                ea = plsc.load_gather(rbuf, [src])
                el = plsc.load_gather(llbuf, [pj])
                contrib = ea * wv[sl] * el * ivec
                plsc.addupdate_scatter(wbuf, [dst], contrib)

    def pow2_scale(accv):
        for sh in (8, 4, 2, 1):
            perm = jnp.bitwise_and(lanes + sh, L - 1)
            accv = jnp.maximum(
                accv, accv.at[perm].get(mode="promise_in_bounds"))
        ebits = jnp.bitwise_and(
            lax.bitcast_convert_type(accv, jnp.int32), 0x7F800000)
        s_vec = lax.bitcast_convert_type(ebits, jnp.float32)
        i_vec = lax.bitcast_convert_type(0x7F000000 - ebits, jnp.float32)
        return s_vec, i_vec

    def finish(t, rbuf, wbuf, n_chunks, scal_v, inv_v):
        @plsc.parallel_loop(0, n_chunks, unroll=4,
                            carry=jnp.zeros((L,), jnp.float32))
        def _mxz(i, acc):
            sl = pl.ds(i * L, L)
            acc = jnp.maximum(acc, wbuf[sl])
            rbuf[sl] = jnp.zeros((L,), jnp.float32)
            return acc
        s_vec, i_vec = pow2_scale(_mxz)
        plsc.store_scatter(scal_v, [jnp.full((L,), t, jnp.int32)],
                           s_vec, mask=lanes == 0)
        inv_v[...] = i_vec

    def step_pre(t, rbuf, wbuf, llbuf, parity, nrbuf, nwbuf):
        @plsc.parallel_loop(0, C // L, unroll=4)
        def _expll(i):
            sl = pl.ds(i * L, L)
            llbuf[sl] = jnp.exp(jnp.clip(llbuf[sl], -30.0, 30.0))

        ivd = invD_v[...]

        @pl.when(is_a)
        def _():
            arc_pass(0, cA >> 1, 4, dsd_v, dpdf_v, dw_v,
                     rbuf, wbuf, llbuf, ivd)
            pltpu.sync_copy(wbuf.at[pl.ds(0, HALF)],
                            spmA.at[pl.ds((p * 2 + parity) * HALF, HALF)])

        @pl.when(is_b)
        def _():
            arc_pass(startB >> 1, (startB + cB) >> 1, 4, dsd_v, dpdf_v,
                     dw_v, rbuf, wbuf, llbuf, ivd)
            pltpu.sync_copy(wbuf.at[pl.ds(HALF, REST)],
                            spmB.at[pl.ds((p * 2 + parity) * REST, REST)])
            ivn = invN_v[...]
            arc_pass(0, E_NUM // (2 * L), 2, nsd_v, npdf_v, nw_v,
                     nrbuf, nwbuf, llbuf, ivn)
            finish(t, nrbuf, nwbuf, S_NPAD // L, scalN_v, invN_v)

    def step_post(t, rbuf, wbuf, parity):
        @pl.when(is_a)
        def _():
            pltpu.sync_copy(spmB.at[pl.ds((p * 2 + parity) * REST, REST)],
                            wbuf.at[pl.ds(HALF, REST)])

        @pl.when(is_b)
        def _():
            pltpu.sync_copy(spmA.at[pl.ds((p * 2 + parity) * HALF, HALF)],
                            wbuf.at[pl.ds(0, HALF)])

        finish(t, rbuf, wbuf, S_PAD // L, scalD_v, invD_v)

    def k_body(k, carry):
        t0 = 2 * k
        t1 = t0 + 1
        t2 = t0 + 2

        @pl.when(t1 < seqlen)
        def _():
            pltpu.async_copy(ll_h.at[u, t1], llB, semB)

        @pl.when(t0 < seqlen)
        def _():
            pltpu.make_async_copy(ll_h.at[u, 0], llA, semA).wait()
            step_pre(t0, bufDA, bufDB, llA, 0, bufNA, bufNB)
        plsc.subcore_barrier()

        @pl.when(t0 < seqlen)
        def _():
            step_post(t0, bufDA, bufDB, 0)

        @pl.when(t2 < seqlen)
        def _():
            pltpu.async_copy(ll_h.at[u, t2], llA, semA)

        @pl.when(t1 < seqlen)
        def _():
            pltpu.make_async_copy(ll_h.at[u, 0], llB, semB).wait()
            step_pre(t1, bufDB, bufDA, llB, 1, bufNB, bufNA)
        plsc.subcore_barrier()

        @pl.when(t1 < seqlen)
        def _():
            step_post(t1, bufDB, bufDA, 1)
        return carry

    lax.fori_loop(0, T // 2, k_body, 0)

    even = jnp.bitwise_and(seqlen, 1) == 0
    fD = invD_v[...]

    @pl.when(is_a)
    def _():
        @pl.when(even)
        def _():
            @plsc.parallel_loop(0, S_PAD // L, unroll=4)
            def _oA(i):
                sl = pl.ds(i * L, L)
                bufDA[sl] = bufDA[sl] * fD

        @pl.when(jnp.logical_not(even))
        def _():
            @plsc.parallel_loop(0, S_PAD // L, unroll=4)
            def _oB(i):
                sl = pl.ds(i * L, L)
                bufDA[sl] = bufDB[sl] * fD

        pltpu.sync_copy(bufDA, ealpha_o.at[u])
        pltpu.sync_copy(scalD_v, scales_o.at[u])

    @pl.when(is_b)
    def _():
        fN = invN_v[...]

        @plsc.parallel_loop(0, S_PAD // L, unroll=4)
        def _z(i):
            llA[pl.ds(i * L, L)] = jnp.zeros((L,), jnp.float32)

        @pl.when(even)
        def _():
            @plsc.parallel_loop(0, S_NPAD // L)
            def _oNA(i):
                sl = pl.ds(i * L, L)
                llA[sl] = bufNA[sl] * fN

        @pl.when(jnp.logical_not(even))
        def _():
            @plsc.parallel_loop(0, S_NPAD // L)
            def _oNB(i):
                sl = pl.ds(i * L, L)
                llA[sl] = bufNB[sl] * fN

        pltpu.sync_copy(llA, ealpha_o.at[16 + u])
        pltpu.sync_copy(scalN_v, scales_o.at[16 + u])


def _tc_combine_body(scales_ref, ealpha_ref, finals_ref, out_ref):
    logs = jnp.log(scales_ref[...])
    acc = jnp.sum(logs, axis=1, keepdims=True)
    ef = jnp.exp(finals_ref[...])
    ef_rows = jnp.concatenate(
        [jnp.broadcast_to(ef[0:1, :], (16, S_PAD)),
         jnp.broadcast_to(ef[1:2, :], (16, S_PAD))], axis=0)
    mass = jnp.sum(ealpha_ref[...] * ef_rows, axis=1, keepdims=True)
    llh = acc + jnp.log(mass + 1e-30)
    sign = jnp.where(
        lax.broadcasted_iota(jnp.int32, (32, 1), 0) < 16, 1.0, -1.0)
    out_ref[...] = jnp.broadcast_to(jnp.sum(llh * sign), (1, 1))


def kernel(input, seqlengths, num_src, num_dst, num_pdf, num_weight,
           num_start, num_final, den_src, den_dst, den_pdf, den_weight,
           den_start, den_final):
    order = jnp.argsort(den_src, stable=True)
    den_src = den_src[order]
    den_dst = den_dst[order]
    den_pdf = den_pdf[order]
    den_weight = den_weight[order]
    key = (den_dst >= HALF).astype(jnp.int32)
    n1 = jnp.sum(key)
    n0 = E_DEN - n1
    g0 = ((n0 + 127) // 128) * 128
    r0 = jnp.cumsum(1 - key) - 1
    r1 = jnp.cumsum(key) - 1
    pos = jnp.where(key == 0, r0, g0 + r1)
    dsd = jnp.zeros((CAP_DEN,), jnp.int32).at[pos].set(
        den_src | (den_dst << 16))
    dpdf = jnp.zeros((CAP_DEN,), jnp.int32).at[pos].set(den_pdf)
    dw = jnp.full((CAP_DEN,), NEG, jnp.float32).at[pos].set(den_weight)
    cA = g0 // 16
    startB = g0 // 16
    cB = ((n1 + 127) // 128) * 8
    params = (jnp.zeros((16,), jnp.int32)
              .at[0].set(cA).at[1].set(startB).at[2].set(cB))

    def ilv16(x):
        return x.reshape(-1, 2, L).transpose(0, 2, 1).reshape(-1).astype(
            jnp.int16)

    nsd = num_src | (num_dst << 16)
    nstart = jnp.pad(num_start, (0, S_NPAD - num_start.shape[0]),
                     constant_values=NEG)
    dstart = jnp.pad(den_start, (0, S_PAD - den_start.shape[0]),
                     constant_values=NEG)
    finals2 = jnp.stack([
        jnp.pad(den_final, (0, S_PAD - den_final.shape[0]),
                constant_values=NEG),
        jnp.pad(num_final, (0, S_PAD - num_final.shape[0]),
                constant_values=NEG)])

    mesh = plsc.VectorSubcoreMesh(core_axis_name="c", subcore_axis_name="s")
    sc_fwd = pl.kernel(
        _sc_forward_body,
        out_type=(jax.ShapeDtypeStruct((32, S_PAD), jnp.float32),
                  jax.ShapeDtypeStruct((32, T_PAD), jnp.float32)),
        mesh=mesh,
        compiler_params=pltpu.CompilerParams(needs_layout_passes=False),
        scratch_types=[
            pltpu.VMEM((CAP_DEN,), jnp.int32),
            pltpu.VMEM((CAP_DEN,), jnp.int16),
            pltpu.VMEM((CAP_DEN,), jnp.float32),
            pltpu.VMEM((E_NUM,), jnp.int32),
            pltpu.VMEM((E_NUM,), jnp.int16),
            pltpu.VMEM((E_NUM,), jnp.float32),
            pltpu.VMEM((S_PAD,), jnp.float32),
            pltpu.VMEM((S_PAD,), jnp.float32),
            pltpu.VMEM((S_NPAD,), jnp.float32),
            pltpu.VMEM((S_NPAD,), jnp.float32),
            pltpu.VMEM((C,), jnp.float32),
            pltpu.VMEM((C,), jnp.float32),
            pltpu.VMEM((L,), jnp.int32),
            pltpu.VMEM((L,), jnp.int32),
            pltpu.VMEM((T_PAD,), jnp.float32),
            pltpu.VMEM((T_PAD,), jnp.float32),
            pltpu.VMEM((L,), jnp.float32),
            pltpu.VMEM((L,), jnp.float32),
            pltpu.VMEM_SHARED((8 * 2 * HALF,), jnp.float32),
            pltpu.VMEM_SHARED((8 * 2 * REST,), jnp.float32),
            pltpu.SemaphoreType.DMA,
            pltpu.SemaphoreType.DMA,
        ],
    )
    ealpha32, scales32 = sc_fwd(input, seqlengths, params, dsd,
                                ilv16(dpdf), dw, nsd, ilv16(num_pdf),
                                num_weight, dstart, nstart)

    loss11 = pl.pallas_call(
        _tc_combine_body,
        out_shape=jax.ShapeDtypeStruct((1, 1), jnp.float32),
    )(scales32, ealpha32, finals2)
    return loss11[0, 0]


# speedup vs baseline: 1.0222x; 1.0222x over previous
import jax
import jax.numpy as jnp
from jax import lax
from jax.experimental import pallas as pl
from jax.experimental.pallas import tpu as pltpu
from jax.experimental.pallas import tpu_sc as plsc

B = 16
T = 300
C = 2048
S_PAD = 2048
S_NPAD = 208
E_DEN = 16000
E_NUM = 1600
CAP_DEN = 16256
HALF = 1112
REST = 888
T_PAD = 304
NEG = -1e5
L = 16


def _sc_forward_body(ll_h, seql_h, prm_h, dsd_h, dpdf_h, dw_h,
                     nsd_h, npdf_h, nw_h, dstart_h, nstart_h,
                     ealpha_o, scales_o,
                     dsd_v, dpdf_v, dw_v, nsd_v, npdf_v, nw_v,
                     bufDA, bufDB, bufNA, bufNB, llA, llB,
                     seql_v, prm_v, scalD_v, scalN_v, invD_v, invN_v,
                     spmA, spmB, semA, semB):
    c = lax.axis_index("c")
    s = lax.axis_index("s")
    p = lax.shift_right_logical(s, 1)
    h = jnp.bitwise_and(s, 1)
    u = c * 8 + p
    is_a = h == 0
    is_b = h == 1
    lanes = lax.iota(jnp.int32, L)

    pltpu.sync_copy(seql_h, seql_v)
    pltpu.sync_copy(prm_h, prm_v)
    pltpu.sync_copy(dsd_h, dsd_v)
    pltpu.sync_copy(dpdf_h, dpdf_v)
    pltpu.sync_copy(dw_h, dw_v)
    pltpu.sync_copy(dstart_h, bufDA)

    @pl.when(is_b)
    def _():
        pltpu.sync_copy(nsd_h, nsd_v)
        pltpu.sync_copy(npdf_h, npdf_v)
        pltpu.sync_copy(nw_h, nw_v)
        pltpu.sync_copy(nstart_h, bufNA)

    seqlen = seql_v[...].at[jnp.full((L,), u, jnp.int32)].get(
        mode="promise_in_bounds")[0]
    prm = prm_v[...]
    cA = prm[0]
    startB = prm[1]
    cB = prm[2]

    @plsc.parallel_loop(0, CAP_DEN // L, unroll=4)
    def _expwd(i):
        sl = pl.ds(i * L, L)
        dw_v[sl] = jnp.exp(dw_v[sl])

    @plsc.parallel_loop(0, S_PAD // L, unroll=4)
    def _initD(i):
        sl = pl.ds(i * L, L)
        bufDA[sl] = jnp.exp(bufDA[sl])
        bufDB[sl] = jnp.zeros((L,), jnp.float32)

    @plsc.parallel_loop(0, T_PAD // L, unroll=4)
    def _ones(i):
        sl = pl.ds(i * L, L)
        scalD_v[sl] = jnp.full((L,), 1.0, jnp.float32)
        scalN_v[sl] = jnp.full((L,), 1.0, jnp.float32)

    @pl.when(is_b)
    def _():
        @plsc.parallel_loop(0, E_NUM // L, unroll=4)
        def _expwn(i):
            sl = pl.ds(i * L, L)
            nw_v[sl] = jnp.exp(nw_v[sl])

        @plsc.parallel_loop(0, S_NPAD // L)
        def _initN(i):
            sl = pl.ds(i * L, L)
            bufNA[sl] = jnp.exp(bufNA[sl])
            bufNB[sl] = jnp.zeros((L,), jnp.float32)

    invD_v[...] = jnp.full((L,), 1.0, jnp.float32)
    invN_v[...] = jnp.full((L,), 1.0, jnp.float32)
    pltpu.async_copy(ll_h.at[u, 0], llA, semA)

    def arc_pass(lo2, hi2, unroll, sdv, pdfv, wv, rbuf, wbuf, llbuf, ivec):
        @plsc.parallel_loop(lo2, hi2, unroll=unroll)
        def _arcs(i):
            base = i * 2 * L
            pp = pdfv[pl.ds(base, 2 * L)]
            p0, p1 = plsc.unpack(pp, format=plsc.PackFormat.INTERLEAVED,
                                 preferred_element_type=jnp.int32)
            for j, pj in ((0, p0), (1, p1)):
                sl = pl.ds(base + j * L, L)
                sd = sdv[sl]
                src = jnp.bitwise_and(sd, 0xFFFF)
                dst = lax.shift_right_logical(sd, 16)
                ea = plsc.load_gather(rbuf, [src])
                el = plsc.load_gather(llbuf, [pj])
                contrib = ea * wv[sl] * el * ivec
                plsc.addupdate_scatter(wbuf, [dst], contrib)

    def pow2_scale(accv):
        for sh in (8, 4, 2, 1):
            perm = jnp.bitwise_and(lanes + sh, L - 1)
            accv = jnp.maximum(
                accv, accv.at[perm].get(mode="promise_in_bounds"))
        ebits = jnp.bitwise_and(
            lax.bitcast_convert_type(accv, jnp.int32), 0x7F800000)
        s_vec = lax.bitcast_convert_type(ebits, jnp.float32)
        i_vec = lax.bitcast_convert_type(0x7F000000 - ebits, jnp.float32)
        return s_vec, i_vec

    def finish(t, rbuf, wbuf, n_chunks, scal_v, inv_v, unroll=4):
        @plsc.parallel_loop(0, n_chunks, unroll=unroll,
                            carry=jnp.zeros((L,), jnp.float32))
        def _mxz(i, acc):
            sl = pl.ds(i * L, L)
            acc = jnp.maximum(acc, wbuf[sl])
            rbuf[sl] = jnp.zeros((L,), jnp.float32)
            return acc
        s_vec, i_vec = pow2_scale(_mxz)
        plsc.store_scatter(scal_v, [jnp.full((L,), t, jnp.int32)],
                           s_vec, mask=lanes == 0)
        inv_v[...] = i_vec

    def step_pre(t, rbuf, wbuf, llbuf, parity, nrbuf, nwbuf):
        @plsc.parallel_loop(0, C // L, unroll=8)
        def _expll(i):
            sl = pl.ds(i * L, L)
            llbuf[sl] = jnp.exp(jnp.clip(llbuf[sl], -30.0, 30.0))

        ivd = invD_v[...]

        @pl.when(is_a)
        def _():
            arc_pass(0, cA >> 1, 4, dsd_v, dpdf_v, dw_v,
                     rbuf, wbuf, llbuf, ivd)
            pltpu.sync_copy(wbuf.at[pl.ds(0, HALF)],
                            spmA.at[pl.ds((p * 2 + parity) * HALF, HALF)])

        @pl.when(is_b)
        def _():
            arc_pass(startB >> 1, (startB + cB) >> 1, 4, dsd_v, dpdf_v,
                     dw_v, rbuf, wbuf, llbuf, ivd)
            pltpu.sync_copy(wbuf.at[pl.ds(HALF, REST)],
                            spmB.at[pl.ds((p * 2 + parity) * REST, REST)])
            ivn = invN_v[...]
            arc_pass(0, E_NUM // (2 * L), 2, nsd_v, npdf_v, nw_v,
                     nrbuf, nwbuf, llbuf, ivn)
            finish(t, nrbuf, nwbuf, S_NPAD // L, scalN_v, invN_v,
                   unroll=1)

    def step_post(t, rbuf, wbuf, parity):
        @pl.when(is_a)
        def _():
            pltpu.sync_copy(spmB.at[pl.ds((p * 2 + parity) * REST, REST)],
                            wbuf.at[pl.ds(HALF, REST)])

        @pl.when(is_b)
        def _():
            pltpu.sync_copy(spmA.at[pl.ds((p * 2 + parity) * HALF, HALF)],
                            wbuf.at[pl.ds(0, HALF)])

        finish(t, rbuf, wbuf, S_PAD // L, scalD_v, invD_v, unroll=8)

    def k_body(k, carry):
        t0 = 2 * k
        t1 = t0 + 1
        t2 = t0 + 2

        @pl.when(t1 < seqlen)
        def _():
            pltpu.async_copy(ll_h.at[u, t1], llB, semB)

        @pl.when(t0 < seqlen)
        def _():
            pltpu.make_async_copy(ll_h.at[u, 0], llA, semA).wait()
            step_pre(t0, bufDA, bufDB, llA, 0, bufNA, bufNB)
        plsc.subcore_barrier()

        @pl.when(t0 < seqlen)
        def _():
            step_post(t0, bufDA, bufDB, 0)

        @pl.when(t2 < seqlen)
        def _():
            pltpu.async_copy(ll_h.at[u, t2], llA, semA)

        @pl.when(t1 < seqlen)
        def _():
            pltpu.make_async_copy(ll_h.at[u, 0], llB, semB).wait()
            step_pre(t1, bufDB, bufDA, llB, 1, bufNB, bufNA)
        plsc.subcore_barrier()

        @pl.when(t1 < seqlen)
        def _():
            step_post(t1, bufDB, bufDA, 1)
        return carry

    lax.fori_loop(0, T // 2, k_body, 0)

    even = jnp.bitwise_and(seqlen, 1) == 0
    fD = invD_v[...]

    @pl.when(is_a)
    def _():
        @pl.when(even)
        def _():
            @plsc.parallel_loop(0, S_PAD // L, unroll=4)
            def _oA(i):
                sl = pl.ds(i * L, L)
                bufDA[sl] = bufDA[sl] * fD

        @pl.when(jnp.logical_not(even))
        def _():
            @plsc.parallel_loop(0, S_PAD // L, unroll=4)
            def _oB(i):
                sl = pl.ds(i * L, L)
                bufDA[sl] = bufDB[sl] * fD

        pltpu.sync_copy(bufDA, ealpha_o.at[u])
        pltpu.sync_copy(scalD_v, scales_o.at[u])

    @pl.when(is_b)
    def _():
        fN = invN_v[...]

        @plsc.parallel_loop(0, S_PAD // L, unroll=4)
        def _z(i):
            llA[pl.ds(i * L, L)] = jnp.zeros((L,), jnp.float32)

        @pl.when(even)
        def _():
            @plsc.parallel_loop(0, S_NPAD // L)
            def _oNA(i):
                sl = pl.ds(i * L, L)
                llA[sl] = bufNA[sl] * fN

        @pl.when(jnp.logical_not(even))
        def _():
            @plsc.parallel_loop(0, S_NPAD // L)
            def _oNB(i):
                sl = pl.ds(i * L, L)
                llA[sl] = bufNB[sl] * fN

        pltpu.sync_copy(llA, ealpha_o.at[16 + u])
        pltpu.sync_copy(scalN_v, scales_o.at[16 + u])


def _tc_combine_body(scales_ref, ealpha_ref, finals_ref, out_ref):
    logs = jnp.log(scales_ref[...])
    acc = jnp.sum(logs, axis=1, keepdims=True)
    ef = jnp.exp(finals_ref[...])
    ef_rows = jnp.concatenate(
        [jnp.broadcast_to(ef[0:1, :], (16, S_PAD)),
         jnp.broadcast_to(ef[1:2, :], (16, S_PAD))], axis=0)
    mass = jnp.sum(ealpha_ref[...] * ef_rows, axis=1, keepdims=True)
    llh = acc + jnp.log(mass + 1e-30)
    sign = jnp.where(
        lax.broadcasted_iota(jnp.int32, (32, 1), 0) < 16, 1.0, -1.0)
    out_ref[...] = jnp.broadcast_to(jnp.sum(llh * sign), (1, 1))


def kernel(input, seqlengths, num_src, num_dst, num_pdf, num_weight,
           num_start, num_final, den_src, den_dst, den_pdf, den_weight,
           den_start, den_final):
    order = jnp.argsort(den_src, stable=True)
    den_src = den_src[order]
    den_dst = den_dst[order]
    den_pdf = den_pdf[order]
    den_weight = den_weight[order]
    key = (den_dst >= HALF).astype(jnp.int32)
    n1 = jnp.sum(key)
    n0 = E_DEN - n1
    g0 = ((n0 + 127) // 128) * 128
    r0 = jnp.cumsum(1 - key) - 1
    r1 = jnp.cumsum(key) - 1
    pos = jnp.where(key == 0, r0, g0 + r1)
    dsd = jnp.zeros((CAP_DEN,), jnp.int32).at[pos].set(
        den_src | (den_dst << 16))
    dpdf = jnp.zeros((CAP_DEN,), jnp.int32).at[pos].set(den_pdf)
    dw = jnp.full((CAP_DEN,), NEG, jnp.float32).at[pos].set(den_weight)
    cA = g0 // 16
    startB = g0 // 16
    cB = ((n1 + 127) // 128) * 8
    params = (jnp.zeros((16,), jnp.int32)
              .at[0].set(cA).at[1].set(startB).at[2].set(cB))

    def ilv16(x):
        return x.reshape(-1, 2, L).transpose(0, 2, 1).reshape(-1).astype(
            jnp.int16)

    nsd = num_src | (num_dst << 16)
    nstart = jnp.pad(num_start, (0, S_NPAD - num_start.shape[0]),
                     constant_values=NEG)
    dstart = jnp.pad(den_start, (0, S_PAD - den_start.shape[0]),
                     constant_values=NEG)
    finals2 = jnp.stack([
        jnp.pad(den_final, (0, S_PAD - den_final.shape[0]),
                constant_values=NEG),
        jnp.pad(num_final, (0, S_PAD - num_final.shape[0]),
                constant_values=NEG)])

    mesh = plsc.VectorSubcoreMesh(core_axis_name="c", subcore_axis_name="s")
    sc_fwd = pl.kernel(
        _sc_forward_body,
        out_type=(jax.ShapeDtypeStruct((32, S_PAD), jnp.float32),
                  jax.ShapeDtypeStruct((32, T_PAD), jnp.float32)),
        mesh=mesh,
        compiler_params=pltpu.CompilerParams(needs_layout_passes=False),
        scratch_types=[
            pltpu.VMEM((CAP_DEN,), jnp.int32),
            pltpu.VMEM((CAP_DEN,), jnp.int16),
            pltpu.VMEM((CAP_DEN,), jnp.float32),
            pltpu.VMEM((E_NUM,), jnp.int32),
            pltpu.VMEM((E_NUM,), jnp.int16),
            pltpu.VMEM((E_NUM,), jnp.float32),
            pltpu.VMEM((S_PAD,), jnp.float32),
            pltpu.VMEM((S_PAD,), jnp.float32),
            pltpu.VMEM((S_NPAD,), jnp.float32),
            pltpu.VMEM((S_NPAD,), jnp.float32),
            pltpu.VMEM((C,), jnp.float32),
            pltpu.VMEM((C,), jnp.float32),
            pltpu.VMEM((L,), jnp.int32),
            pltpu.VMEM((L,), jnp.int32),
            pltpu.VMEM((T_PAD,), jnp.float32),
            pltpu.VMEM((T_PAD,), jnp.float32),
            pltpu.VMEM((L,), jnp.float32),
            pltpu.VMEM((L,), jnp.float32),
            pltpu.VMEM_SHARED((8 * 2 * HALF,), jnp.float32),
            pltpu.VMEM_SHARED((8 * 2 * REST,), jnp.float32),
            pltpu.SemaphoreType.DMA,
            pltpu.SemaphoreType.DMA,
        ],
    )
    ealpha32, scales32 = sc_fwd(input, seqlengths, params, dsd,
                                ilv16(dpdf), dw, nsd, ilv16(num_pdf),
                                num_weight, dstart, nstart)

    loss11 = pl.pallas_call(
        _tc_combine_body,
        out_shape=jax.ShapeDtypeStruct((1, 1), jnp.float32),
    )(scales32, ealpha32, finals2)
    return loss11[0, 0]
